# baseline (speedup 1.0000x reference)
import functools

import jax
import jax.numpy as jnp
from jax import lax
from jax.experimental import pallas as pl
from jax.experimental.pallas import tpu as pltpu
from jax.experimental.pallas import tpu_sc as plsc

N_NODES = 10000
E_EDGES = 320000
FDIM = 128

NC = 2
NS = 16
NW = NC * NS
CK = 128
RCH = 16
NRND = 5
NCH = RCH * NRND
E_PAD = NW * CK * NCH
ACC_ROWS = 10240
ROWS_PER_TILE = ACC_ROWS // NS


@functools.partial(
    pl.kernel,
    out_type=jax.ShapeDtypeStruct((NC, ACC_ROWS, FDIM), jnp.float32),
    mesh=plsc.VectorSubcoreMesh(
        core_axis_name="c", subcore_axis_name="s", num_cores=NC, num_subcores=NS
    ),
    scratch_types=[
        pltpu.VMEM((NCH, CK), jnp.int32),
        pltpu.VMEM((NCH, CK), jnp.int32),
        pltpu.VMEM((CK, FDIM), jnp.float32),
        pltpu.VMEM_SHARED((ACC_ROWS, FDIM), jnp.float32),
        pltpu.SemaphoreType.DMA,
    ],
)
def _sc_aggregate(table_h, src_h, dst_h, out_h, src_v, dst_v, rows_a, acc_s, sem_a):
    c = lax.axis_index("c")
    s = lax.axis_index("s")
    wid = s * NC + c

    zvec = jnp.zeros((16,), jnp.float32)

    def zb_body(k, carry):
        rows_a[k // 8, pl.ds((k % 8) * 16, 16)] = zvec
        return carry

    lax.fori_loop(0, CK * 8, zb_body, 0)

    base = s * ROWS_PER_TILE

    def zc_body(k, carry):
        pltpu.sync_copy(rows_a, acc_s.at[pl.ds(base + k * CK, CK)])
        return carry

    lax.fori_loop(0, ROWS_PER_TILE // CK, zc_body, 0)

    plsc.subcore_barrier()

    pltpu.sync_copy(src_h.at[wid], src_v)
    pltpu.sync_copy(dst_h.at[wid], dst_v)

    def pipe_body(i, carry):
        pltpu.async_copy(table_h.at[src_v.at[i]], rows_a, sem_a).wait()
        pltpu.sync_copy(rows_a, acc_s.at[dst_v.at[i]], add=True)
        return carry

    lax.fori_loop(0, NCH, pipe_body, 0)
    plsc.subcore_barrier()

    pltpu.sync_copy(
        acc_s.at[pl.ds(base, ROWS_PER_TILE)],
        out_h.at[c, pl.ds(base, ROWS_PER_TILE)],
    )


def _combine_body(p0_ref, p1_ref, x_ref, wr_ref, wt_ref, b_ref, o_ref):
    agg = p0_ref[0] + p1_ref[0]
    o_ref[...] = (
        jnp.dot(agg, wr_ref[...], preferred_element_type=jnp.float32)
        + jnp.dot(x_ref[...], wt_ref[...], preferred_element_type=jnp.float32)
        + b_ref[...]
    )


_BR = 1000


def _combine(parts, x, w_rel, w_root, b):
    return pl.pallas_call(
        _combine_body,
        grid=(N_NODES // _BR,),
        in_specs=[
            pl.BlockSpec((1, _BR, FDIM), lambda i: (0, i, 0)),
            pl.BlockSpec((1, _BR, FDIM), lambda i: (1, i, 0)),
            pl.BlockSpec((_BR, FDIM), lambda i: (i, 0)),
            pl.BlockSpec((FDIM, FDIM), lambda i: (0, 0)),
            pl.BlockSpec((FDIM, FDIM), lambda i: (0, 0)),
            pl.BlockSpec((1, FDIM), lambda i: (0, 0)),
        ],
        out_specs=pl.BlockSpec((_BR, FDIM), lambda i: (i, 0)),
        out_shape=jax.ShapeDtypeStruct((N_NODES, FDIM), jnp.float32),
    )(parts, parts, x, w_rel, w_root, b)


def kernel(x, edge_index, W_rel1, b_rel1, W_root1, W_rel2, b_rel2, W_root2):
    pad = E_PAD - E_EDGES
    src = jnp.concatenate([edge_index[0], jnp.zeros((pad,), edge_index.dtype)])
    spare = ACC_ROWS - N_NODES
    pad_dst = (N_NODES + jnp.arange(pad, dtype=edge_index.dtype) % spare)
    dst = jnp.concatenate([edge_index[1], pad_dst])
    src_r = src.reshape(NCH, CK, NW).transpose(2, 0, 1)
    dst_r = dst.reshape(NCH, CK, NW).transpose(2, 0, 1)

    b1 = b_rel1.reshape(1, FDIM)
    b2 = b_rel2.reshape(1, FDIM)

    parts1 = _sc_aggregate(x, src_r, dst_r)
    h = _combine(parts1, x, W_rel1, W_root1, b1)
    parts2 = _sc_aggregate(h, src_r, dst_r)
    return _combine(parts2, h, W_rel2, W_root2, b2)


# speedup vs baseline: 1.4661x; 1.4661x over previous
import functools

import jax
import jax.numpy as jnp
from jax import lax
from jax.experimental import pallas as pl
from jax.experimental.pallas import tpu as pltpu
from jax.experimental.pallas import tpu_sc as plsc

N_NODES = 10000
E_EDGES = 320000
FDIM = 128

NC = 2
NS = 16
NW = NC * NS
CK = 128
NCH = -(-E_EDGES // (NW * CK))
E_PAD = NW * CK * NCH
ACC_ROWS = 10240
ZROWS = 64
ROWS_PER_TILE = ACC_ROWS // NS


@functools.partial(
    pl.kernel,
    out_type=jax.ShapeDtypeStruct((NC, ACC_ROWS, FDIM), jnp.float32),
    mesh=plsc.VectorSubcoreMesh(
        core_axis_name="c", subcore_axis_name="s", num_cores=NC, num_subcores=NS
    ),
    scratch_types=[
        pltpu.VMEM((NCH, CK), jnp.int32),
        pltpu.VMEM((NCH, CK), jnp.int32),
        pltpu.VMEM((CK, FDIM), jnp.float32),
        pltpu.VMEM((ZROWS, FDIM), jnp.float32),
        pltpu.VMEM_SHARED((ACC_ROWS, FDIM), jnp.float32),
        pltpu.SemaphoreType.DMA,
    ],
)
def _sc_aggregate(table_h, src_h, dst_h, out_h, src_v, dst_v, rows_v, zbuf_v, acc_s, sem):
    c = lax.axis_index("c")
    s = lax.axis_index("s")
    wid = s * NC + c

    zvec = jnp.zeros((16,), jnp.float32)

    def zb_body(k, carry):
        zbuf_v[k // 8, pl.ds((k % 8) * 16, 16)] = zvec
        return carry

    lax.fori_loop(0, ZROWS * 8, zb_body, 0)

    base = s * ROWS_PER_TILE

    def zc_body(k, carry):
        pltpu.sync_copy(zbuf_v, acc_s.at[pl.ds(base + k * ZROWS, ZROWS)])
        return carry

    lax.fori_loop(0, ROWS_PER_TILE // ZROWS, zc_body, 0)
    plsc.subcore_barrier()

    pltpu.sync_copy(src_h.at[wid], src_v)
    pltpu.sync_copy(dst_h.at[wid], dst_v)

    def edge_body(j, carry):
        pltpu.async_copy(table_h.at[src_v.at[j]], rows_v, sem).wait()
        pltpu.sync_copy(rows_v, acc_s.at[dst_v.at[j]], add=True)
        return carry

    lax.fori_loop(0, NCH, edge_body, 0)
    plsc.subcore_barrier()

    pltpu.sync_copy(
        acc_s.at[pl.ds(base, ROWS_PER_TILE)],
        out_h.at[c, pl.ds(base, ROWS_PER_TILE)],
    )


def _combine_body(p0_ref, p1_ref, x_ref, wr_ref, wt_ref, b_ref, o_ref):
    agg = p0_ref[0] + p1_ref[0]
    o_ref[...] = (
        jnp.dot(agg, wr_ref[...], preferred_element_type=jnp.float32)
        + jnp.dot(x_ref[...], wt_ref[...], preferred_element_type=jnp.float32)
        + b_ref[...]
    )


_BR = 1000


def _combine(parts, x, w_rel, w_root, b):
    return pl.pallas_call(
        _combine_body,
        grid=(N_NODES // _BR,),
        in_specs=[
            pl.BlockSpec((1, _BR, FDIM), lambda i: (0, i, 0)),
            pl.BlockSpec((1, _BR, FDIM), lambda i: (1, i, 0)),
            pl.BlockSpec((_BR, FDIM), lambda i: (i, 0)),
            pl.BlockSpec((FDIM, FDIM), lambda i: (0, 0)),
            pl.BlockSpec((FDIM, FDIM), lambda i: (0, 0)),
            pl.BlockSpec((1, FDIM), lambda i: (0, 0)),
        ],
        out_specs=pl.BlockSpec((_BR, FDIM), lambda i: (i, 0)),
        out_shape=jax.ShapeDtypeStruct((N_NODES, FDIM), jnp.float32),
    )(parts, parts, x, w_rel, w_root, b)


def kernel(x, edge_index, W_rel1, b_rel1, W_root1, W_rel2, b_rel2, W_root2):
    pad = E_PAD - E_EDGES
    src = jnp.concatenate([edge_index[0], jnp.zeros((pad,), edge_index.dtype)])
    dst = jnp.concatenate(
        [edge_index[1], jnp.full((pad,), ACC_ROWS - 1, edge_index.dtype)]
    )
    src_r = src.reshape(NCH, CK, NW).transpose(2, 0, 1)
    dst_r = dst.reshape(NCH, CK, NW).transpose(2, 0, 1)

    b1 = b_rel1.reshape(1, FDIM)
    b2 = b_rel2.reshape(1, FDIM)

    parts1 = _sc_aggregate(x, src_r, dst_r)
    h = _combine(parts1, x, W_rel1, W_root1, b1)
    parts2 = _sc_aggregate(h, src_r, dst_r)
    return _combine(parts2, h, W_rel2, W_root2, b2)
